# baseline (speedup 1.0000x reference)
import jax
import jax.numpy as jnp
from jax import lax
from jax.experimental import pallas as pl
from jax.experimental.pallas import tpu as pltpu
from jax.experimental.pallas import tpu_sc as plsc

B, L, D = 4096, 200, 32
NC, NS = 2, 16
NW = NC * NS
BW = B // NW
LC = 4
NCH = L // LC
V4 = 250000


def _pe_table():
    pos = jnp.arange(L, dtype=jnp.float32).reshape(-1, 1)
    exponent = jnp.arange(0, D, 2, dtype=jnp.float32).reshape(1, -1) / D
    X = pos / jnp.power(10000.0, exponent)
    pe = jnp.zeros((L, D), dtype=jnp.float32)
    pe = pe.at[:, 0::2].set(jnp.sin(X))
    pe = pe.at[:, 1::2].set(jnp.cos(X))
    return pe


def _splat(v):
    return jnp.full((16,), v, jnp.int32)


def _body(xt_hbm, table_hbm, pe_hbm, out_hbm,
          idx_v, pe_v, r0, r1, o0, o1, gs0, gs1, os0, os1):
    rows = [r0, r1]
    outb = [o0, o1]
    gsem = [gs0, gs1]
    osem = [os0, os1]
    wid = lax.axis_index("s") * NC + lax.axis_index("c")
    bw0 = wid * BW

    pltpu.sync_copy(xt_hbm.at[:, pl.ds(bw0, BW)], idx_v)
    pltpu.sync_copy(pe_hbm, pe_v)

    def gather(c, b):
        for li in range(LC):
            pltpu.async_copy(
                table_hbm.at[idx_v.at[c * LC + li]], rows[b].at[li], gsem[b])

    def gather_wait(c, b):
        for li in range(LC):
            pltpu.make_async_copy(
                table_hbm.at[idx_v.at[c * LC + li]],
                rows[b].at[li], gsem[b]).wait()

    def write(c, b):
        return pltpu.async_copy(
            outb[b], out_hbm.at[pl.ds(c * LC, LC), :, wid], osem[b])

    rowc = [lax.iota(jnp.int32, 16) + 16 * g for g in range(BW // 16)]

    def transpose_add(c, b):
        for li in range(LC):
            l = c * LC + li
            prow = _splat(lax.shift_right_logical(l, 2))
            pcol0 = lax.bitwise_and(l, 3) * D
            li_s = _splat(li)

            @plsc.parallel_loop(0, D, 1, unroll=4)
            def dloop(d, li=li, li_s=li_s, prow=prow, pcol0=pcol0, b=b):
                pe_d = plsc.load_gather(pe_v, [prow, _splat(pcol0 + d)])
                d_s = _splat(d)
                dh = lax.shift_right_logical(d, 3)
                dl = lax.bitwise_and(d, 7)
                for g in range(BW // 16):
                    val = plsc.load_gather(rows[b], [li_s, rowc[g], d_s])
                    outb[b][li, dh, dl, pl.ds(g * 16, 16)] = val + pe_d

    gather(0, 0)
    gather(1, 1)

    def step(i, acc):
        for b in range(2):
            c = 2 * i + b
            gather_wait(c, b)

            @pl.when(c >= 2)
            def _(c=c, b=b):
                pltpu.make_async_copy(
                    outb[b], out_hbm.at[pl.ds((c - 2) * LC, LC), :, wid],
                    osem[b]).wait()

            transpose_add(c, b)
            write(c, b)

            @pl.when(c + 2 < NCH)
            def _(c=c, b=b):
                gather(c + 2, b)
        return acc

    lax.fori_loop(0, NCH // 2, step, 0)

    for b in range(2):
        pltpu.make_async_copy(
            outb[b], out_hbm.at[pl.ds((NCH - 2 + b) * LC, LC), :, wid],
            osem[b]).wait()


@jax.jit
def kernel(x, table):
    xt = jnp.swapaxes(x, 0, 1).astype(jnp.int32)
    t4 = lax.optimization_barrier(table.reshape(V4, 128))
    table_lin = t4.reshape(1000000, D)
    pe50 = _pe_table().reshape(L * D // 128, 128)
    mesh = plsc.VectorSubcoreMesh(core_axis_name="c", subcore_axis_name="s")
    out5 = pl.kernel(
        _body,
        out_type=jax.ShapeDtypeStruct((L, D // 8, NW, 8, BW), jnp.float32),
        mesh=mesh,
        scratch_types=[
            pltpu.VMEM((L, BW), jnp.int32),
            pltpu.VMEM((L * D // 128, 128), jnp.float32),
            pltpu.VMEM((LC, BW, D), jnp.float32),
            pltpu.VMEM((LC, BW, D), jnp.float32),
            pltpu.VMEM((LC, D // 8, 8, BW), jnp.float32),
            pltpu.VMEM((LC, D // 8, 8, BW), jnp.float32),
            pltpu.SemaphoreType.DMA,
            pltpu.SemaphoreType.DMA,
            pltpu.SemaphoreType.DMA,
            pltpu.SemaphoreType.DMA,
        ],
        compiler_params=pltpu.CompilerParams(
            use_tc_tiling_on_sc=False, needs_layout_passes=False),
    )(xt, table_lin, pe50)
    return out5.transpose(2, 4, 0, 1, 3).reshape(B, L, D)
